# baseline (speedup 1.0000x reference)
import jax
import jax.numpy as jnp
from jax import lax
from jax.experimental import pallas as pl
from jax.experimental.pallas import tpu as pltpu
from jax.experimental.pallas import tpu_sc as plsc

NUM_WORKERS = 32
CHUNK_ROWS = 4
LANES = 16


def _build(B, S, D, n_table):
    assert B % NUM_WORKERS == 0
    rows_per_w = B // NUM_WORKERS
    assert rows_per_w % (2 * CHUNK_ROWS) == 0
    n_chunks = rows_per_w // CHUNK_ROWS
    CS = CHUNK_ROWS * S
    assert D % LANES == 0
    d_regs = D // LANES
    n_full_groups = S // LANES
    tail = S - n_full_groups * LANES
    slices = []
    off = 0
    while off < CS:
        n = min(128, CS - off)
        slices.append((off, n))
        off += n

    mesh = plsc.VectorSubcoreMesh(core_axis_name="c", subcore_axis_name="s")

    def body(x_hbm, m_hbm, table_hbm, out_hbm,
             xv0, mv0, rows0, xv1, mv1, rows1, outb, gsem0, gsem1):
        wid = lax.axis_index("s") * 2 + lax.axis_index("c")
        bufs = ((xv0, mv0, rows0, gsem0), (xv1, mv1, rows1, gsem1))

        def load_idx(c, buf):
            xv, mv, _, _ = buf
            base = (wid * rows_per_w + c * CHUNK_ROWS) * S
            pltpu.sync_copy(x_hbm.at[pl.ds(base, CS)], xv)
            pltpu.sync_copy(m_hbm.at[pl.ds(base, CS)], mv.at[pl.ds(0, CS)])

        def gather_copies(buf):
            xv, _, rows_v, gsem = buf
            for off, n in slices:
                yield pltpu.make_async_copy(
                    table_hbm.at[xv.at[pl.ds(off, n)]],
                    rows_v.at[pl.ds(off, n)],
                    gsem,
                )

        def fire(buf):
            for cp in gather_copies(buf):
                cp.start()

        def wait(buf):
            for cp in gather_copies(buf):
                cp.wait()

        def process(c, buf):
            _, mv, rows_v, _ = buf
            row0 = wid * rows_per_w + c * CHUNK_ROWS
            for r in range(CHUNK_ROWS):
                rb = r * S

                def accum_rows(base, mvec, nrows, accs, cnt):
                    out = list(accs)
                    for j in range(nrows):
                        mj = mvec[j]
                        cnt = cnt + mj
                        m = jnp.full((LANES,), mj, jnp.float32)
                        p = (j % 2) * d_regs
                        for d in range(d_regs):
                            out[p + d] = out[p + d] + rows_v[base + j, pl.ds(d * LANES, LANES)] * m
                    return tuple(out), cnt

                def gbody(g, ac):
                    accs, cnt = ac
                    base = rb + g * LANES
                    mvec = mv[pl.ds(base, LANES)]
                    return accum_rows(base, mvec, LANES, accs, cnt)

                z = jnp.zeros((LANES,), jnp.float32)
                accs, cnt = lax.fori_loop(
                    0, n_full_groups, gbody,
                    ((z,) * (2 * d_regs), jnp.float32(0.0)),
                )
                if tail:
                    tbase = rb + n_full_groups * LANES
                    mvec = mv[pl.ds(tbase, LANES)]
                    accs, cnt = accum_rows(tbase, mvec, tail, accs, cnt)
                accs = [accs[d] + accs[d_regs + d] for d in range(d_regs)]
                inv = jnp.float32(1.0) / (
                    jnp.full((LANES,), cnt, jnp.float32) + jnp.float32(1e-9)
                )
                for d in range(d_regs):
                    outb[r, pl.ds(d * LANES, LANES)] = accs[d] * inv

            pltpu.sync_copy(outb, out_hbm.at[pl.ds(row0, CHUNK_ROWS)])

        load_idx(0, bufs[0])
        fire(bufs[0])

        def pair_body(i, carry):
            c0 = 2 * i
            load_idx(c0 + 1, bufs[1])
            fire(bufs[1])
            wait(bufs[0])
            process(c0, bufs[0])

            @pl.when(c0 + 2 < n_chunks)
            def _():
                load_idx(c0 + 2, bufs[0])
                fire(bufs[0])

            wait(bufs[1])
            process(c0 + 1, bufs[1])
            return carry

        lax.fori_loop(0, n_chunks // 2, pair_body, 0)

    return pl.kernel(
        body,
        out_type=jax.ShapeDtypeStruct((B, D), jnp.float32),
        mesh=mesh,
        compiler_params=pltpu.CompilerParams(use_tc_tiling_on_sc=False),
        scratch_types=[
            pltpu.VMEM((CS,), jnp.int32),
            pltpu.VMEM((CS + LANES,), jnp.float32),
            pltpu.VMEM((CS, D), jnp.float32),
            pltpu.VMEM((CS,), jnp.int32),
            pltpu.VMEM((CS + LANES,), jnp.float32),
            pltpu.VMEM((CS, D), jnp.float32),
            pltpu.VMEM((CHUNK_ROWS, D), jnp.float32),
            pltpu.SemaphoreType.DMA,
            pltpu.SemaphoreType.DMA,
        ],
    )


@jax.jit
def kernel(x, mask, table):
    B, S = x.shape
    n_table, D = table.shape
    xf = x.reshape(-1).astype(jnp.int32)
    mf = mask.reshape(-1).astype(jnp.float32)
    return _build(B, S, D, n_table)(xf, mf, table)
